# baseline (speedup 1.0000x reference)
import functools

import jax
import jax.numpy as jnp
from jax import lax
from jax.experimental import pallas as pl
from jax.experimental.pallas import tpu as pltpu
from jax.experimental.pallas import tpu_sc as plsc

NC = 2
NS = 16
NW = NC * NS

SEQ = 200
EMBED = 64
LANES = 16
VPR = EMBED // LANES

SPLIT_A = 104
SPLIT_B = SEQ - SPLIT_A


def _build(batch):
    seqs_per_w = batch // NW
    mesh = plsc.VectorSubcoreMesh(core_axis_name="c", subcore_axis_name="s")

    @functools.partial(
        pl.kernel,
        mesh=mesh,
        compiler_params=pltpu.CompilerParams(use_tc_tiling_on_sc=False),
        out_type=jax.ShapeDtypeStruct((batch, SEQ, EMBED), jnp.float32),
        scratch_types=[
            pltpu.VMEM((SEQ, EMBED), jnp.float32),
            pltpu.VMEM((SEQ,), jnp.int32),
            pltpu.VMEM((SEQ, EMBED), jnp.float32),
            pltpu.SemaphoreType.DMA,
        ],
    )
    def body(x_hbm, tok_hbm, pos_hbm, out_hbm, pos_v, idx_v, rows_v, sem):
        wid = lax.axis_index("s") * NC + lax.axis_index("c")
        base = wid * seqs_per_w
        pltpu.sync_copy(pos_hbm, pos_v)

        def per_seq(i, carry):
            s = base + i
            pltpu.sync_copy(x_hbm.at[s], idx_v)
            cp0 = pltpu.async_copy(
                tok_hbm.at[idx_v.at[pl.ds(0, SPLIT_A)]],
                rows_v.at[pl.ds(0, SPLIT_A)], sem)
            cp1 = pltpu.async_copy(
                tok_hbm.at[idx_v.at[pl.ds(SPLIT_A, SPLIT_B)]],
                rows_v.at[pl.ds(SPLIT_A, SPLIT_B)], sem)
            cp0.wait()
            cp1.wait()

            def add_row(r, c):
                for j in range(VPR):
                    sl = pl.ds(j * LANES, LANES)
                    rows_v[r, sl] = rows_v[r, sl] + pos_v[r, sl]
                return c

            lax.fori_loop(0, SEQ, add_row, 0, unroll=4)
            pltpu.sync_copy(rows_v, out_hbm.at[s])
            return carry

        lax.fori_loop(0, seqs_per_w, per_seq, 0)

    return body


def kernel(x, token_table, pos_table):
    batch = x.shape[0]
    run = _build(batch)
    return run(x.astype(jnp.int32), token_table, pos_table)


# speedup vs baseline: 1.3717x; 1.3717x over previous
import functools

import jax
import jax.numpy as jnp
from jax import lax
from jax.experimental import pallas as pl
from jax.experimental.pallas import tpu as pltpu
from jax.experimental.pallas import tpu_sc as plsc

NC = 2
NS = 16
NW = NC * NS

SEQ = 200
EMBED = 64
LANES = 16
VPR = EMBED // LANES

SPLITS = ((0, 104), (104, 96))

G = 2
NBUF = 3


def _build(batch):
    seqs_per_w = batch // NW
    ngroups = seqs_per_w // G
    mesh = plsc.VectorSubcoreMesh(core_axis_name="c", subcore_axis_name="s")

    @functools.partial(
        pl.kernel,
        mesh=mesh,
        compiler_params=pltpu.CompilerParams(use_tc_tiling_on_sc=False),
        out_type=jax.ShapeDtypeStruct((batch, SEQ, EMBED), jnp.float32),
        scratch_types=[
            pltpu.VMEM((SEQ, EMBED), jnp.float32),
            pltpu.VMEM((NBUF, G, SEQ), jnp.int32),
            pltpu.VMEM((NBUF, G, SEQ, EMBED), jnp.float32),
            pltpu.SemaphoreType.DMA((NBUF,)),
            pltpu.SemaphoreType.DMA((NBUF,)),
        ],
    )
    def body(x_hbm, tok_hbm, pos_hbm, out_hbm, pos_v, idx_v, rows_v, gsems, osems):
        wid = lax.axis_index("s") * NC + lax.axis_index("c")
        base_seq = wid * seqs_per_w
        pltpu.sync_copy(pos_hbm, pos_v)

        def prefill(b):
            def row(r, c):
                for j in range(VPR):
                    sl = pl.ds(j * LANES, LANES)
                    v = pos_v[r, sl]
                    for s in range(G):
                        rows_v[b, s, r, sl] = v
                return c

            lax.fori_loop(0, SEQ, row, 0, unroll=2)

        def issue_gather(g, b):
            s0 = base_seq + g * G
            pltpu.sync_copy(x_hbm.at[pl.ds(s0, G)], idx_v.at[b])
            for s in range(G):
                for (o, n) in SPLITS:
                    pltpu.async_copy(
                        tok_hbm.at[idx_v.at[b, s, pl.ds(o, n)]],
                        rows_v.at[b, s, pl.ds(o, n)],
                        gsems.at[b], add=True)

        def drain_gather(b):
            for s in range(G):
                for (o, n) in SPLITS:
                    pltpu.make_async_copy(
                        tok_hbm.at[idx_v.at[b, s, pl.ds(o, n)]],
                        rows_v.at[b, s, pl.ds(o, n)],
                        gsems.at[b]).wait()

        def issue_out(g, b):
            s0 = base_seq + g * G
            pltpu.async_copy(rows_v.at[b], out_hbm.at[pl.ds(s0, G)], osems.at[b])

        def drain_out(g, b):
            s0 = base_seq + g * G
            pltpu.make_async_copy(
                rows_v.at[b], out_hbm.at[pl.ds(s0, G)], osems.at[b]).wait()

        prefill(0)
        issue_gather(0, 0)

        def step(g, carry):
            b = lax.rem(g, NBUF)
            bn = lax.rem(g + 1, NBUF)

            @pl.when(g >= 2)
            def _():
                drain_out(g - 2, bn)

            @pl.when(g + 1 < ngroups)
            def _():
                prefill(bn)
                issue_gather(g + 1, bn)

            drain_gather(b)
            issue_out(g, b)
            return carry

        lax.fori_loop(0, ngroups, step, 0)

        for g in (ngroups - 2, ngroups - 1):
            drain_out(g, g % NBUF)

    return body


def kernel(x, token_table, pos_table):
    batch = x.shape[0]
    run = _build(batch)
    return run(x.astype(jnp.int32), token_table, pos_table)
